# baseline (speedup 1.0000x reference)
import functools

import jax
import jax.numpy as jnp
from jax import lax
from jax.experimental import pallas as pl
from jax.experimental.pallas import tpu as pltpu
from jax.experimental.pallas import tpu_sc as plsc

BATCH = 16384
D = 32
NC = 2
NS = 16
NW = NC * NS
BPW = BATCH // NW


def _gather_body(u_ids, i_ids, user_tab, item_tab, u_out, i_out,
                 uidx_v, urows_v, iidx_v, irows_v, sem_u, sem_i):
    wid = lax.axis_index("s") * NC + lax.axis_index("c")
    base = wid * BPW
    pltpu.sync_copy(u_ids.at[pl.ds(base, BPW)], uidx_v)
    pltpu.sync_copy(i_ids.at[pl.ds(base, BPW)], iidx_v)
    cu = pltpu.async_copy(user_tab.at[uidx_v], urows_v, sem_u)
    ci = pltpu.async_copy(item_tab.at[iidx_v], irows_v, sem_i)
    cu.wait()
    pltpu.sync_copy(urows_v, u_out.at[pl.ds(base, BPW)])
    ci.wait()
    pltpu.sync_copy(irows_v, i_out.at[pl.ds(base, BPW)])


_sc_gather = pl.kernel(
    _gather_body,
    out_type=(
        jax.ShapeDtypeStruct((BATCH, D), jnp.float32),
        jax.ShapeDtypeStruct((BATCH, D), jnp.float32),
    ),
    mesh=plsc.VectorSubcoreMesh(core_axis_name="c", subcore_axis_name="s"),
    scratch_types=[
        pltpu.VMEM((BPW,), jnp.int32),
        pltpu.VMEM((BPW, D), jnp.float32),
        pltpu.VMEM((BPW,), jnp.int32),
        pltpu.VMEM((BPW, D), jnp.float32),
        pltpu.SemaphoreType.DMA,
        pltpu.SemaphoreType.DMA,
    ],
    compiler_params=pltpu.CompilerParams(use_tc_tiling_on_sc=False),
)


def _mlp_body(u_ref, i_ref, w1u_ref, w1i_ref, b1_ref, w2_ref, b2_ref,
              w3_ref, b3_ref, out_ref):
    u = u_ref[...]
    i = i_ref[...]
    h = jnp.dot(u, w1u_ref[...], preferred_element_type=jnp.float32)
    h = h + jnp.dot(i, w1i_ref[...], preferred_element_type=jnp.float32)
    h = jnp.maximum(h + b1_ref[...], 0.0)
    h2 = jnp.dot(h, w2_ref[...], preferred_element_type=jnp.float32)
    h2 = jnp.maximum(h2 + b2_ref[...], 0.0)
    out_ref[...] = jnp.sum(h2 * w3_ref[...], axis=1, keepdims=True) + b3_ref[...]


_mlp = pl.pallas_call(
    _mlp_body,
    out_shape=jax.ShapeDtypeStruct((BATCH, 1), jnp.float32),
)


def kernel(U_ids, I_ids, user_table, item_table, W1, b1, W2, b2, W3, b3):
    u_ids = U_ids.astype(jnp.int32)
    i_ids = I_ids.astype(jnp.int32)
    u_emb, i_emb = _sc_gather(u_ids, i_ids, user_table, item_table)
    return _mlp(u_emb, i_emb, W1[:D], W1[D:], b1.reshape(1, D),
                W2, b2.reshape(1, 8), W3.reshape(1, 8), b3.reshape(1, 1))


# speedup vs baseline: 1.4923x; 1.4923x over previous
import functools

import jax
import jax.numpy as jnp
from jax import lax
from jax.experimental import pallas as pl
from jax.experimental.pallas import tpu as pltpu
from jax.experimental.pallas import tpu_sc as plsc

BATCH = 16384
D = 32
NC = 2
NS = 16
NW = NC * NS
BPW = BATCH // NW
CH = 128
NCH = BPW // CH


def _gather_body(u_ids, i_ids, user_tab, item_tab, u_out, i_out,
                 idx_v, rows_a, rows_b, sem_a, sem_b):
    wid = lax.axis_index("s") * NC + lax.axis_index("c")
    base = wid * BPW
    pltpu.sync_copy(u_ids.at[pl.ds(base, BPW)], idx_v.at[0])
    pltpu.sync_copy(i_ids.at[pl.ds(base, BPW)], idx_v.at[1])

    bufs = (rows_a, rows_b)
    sems = (sem_a, sem_b)
    tabs = (user_tab, item_tab)
    outs = (u_out, i_out)

    def issue_chunk(step, slot):
        tab, ch = divmod(step, NCH)
        src = tabs[tab]
        buf = bufs[slot]
        sem = sems[slot]

        def body(g, _):
            vec = idx_v[tab, pl.ds(ch * CH + g * 16, 16)]
            off = g * 16
            for l in range(16):
                idx = vec[l]
                pltpu.make_async_copy(src.at[pl.ds(idx, 1)],
                                      buf.at[pl.ds(off + l, 1)], sem).start()
            return _

        lax.fori_loop(0, CH // 16, body, None)

    def drain_chunk(step, slot):
        tab, ch = divmod(step, NCH)
        pltpu.make_async_copy(tabs[tab].at[pl.ds(0, CH)], bufs[slot],
                              sems[slot]).wait()
        pltpu.sync_copy(bufs[slot],
                        outs[tab].at[pl.ds(base + ch * CH, CH)])

    for step in range(2 * NCH + 2):
        slot = step % 2
        if step >= 2:
            drain_chunk(step - 2, slot)
        if step < 2 * NCH:
            issue_chunk(step, slot)


_sc_gather = pl.kernel(
    _gather_body,
    out_type=(
        jax.ShapeDtypeStruct((BATCH, D), jnp.float32),
        jax.ShapeDtypeStruct((BATCH, D), jnp.float32),
    ),
    mesh=plsc.VectorSubcoreMesh(core_axis_name="c", subcore_axis_name="s"),
    scratch_types=[
        pltpu.VMEM((2, BPW), jnp.int32),
        pltpu.VMEM((CH, D), jnp.float32),
        pltpu.VMEM((CH, D), jnp.float32),
        pltpu.SemaphoreType.DMA,
        pltpu.SemaphoreType.DMA,
    ],
)


def _mlp_body(u_ref, i_ref, w1u_ref, w1i_ref, b1_ref, w2_ref, b2_ref,
              w3_ref, b3_ref, out_ref):
    u = u_ref[...]
    i = i_ref[...]
    h = jnp.dot(u, w1u_ref[...], preferred_element_type=jnp.float32)
    h = h + jnp.dot(i, w1i_ref[...], preferred_element_type=jnp.float32)
    h = jnp.maximum(h + b1_ref[...], 0.0)
    h2 = jnp.dot(h, w2_ref[...], preferred_element_type=jnp.float32)
    h2 = jnp.maximum(h2 + b2_ref[...], 0.0)
    out_ref[...] = jnp.sum(h2 * w3_ref[...], axis=1, keepdims=True) + b3_ref[...]


_mlp = pl.pallas_call(
    _mlp_body,
    out_shape=jax.ShapeDtypeStruct((BATCH, 1), jnp.float32),
)


def kernel(U_ids, I_ids, user_table, item_table, W1, b1, W2, b2, W3, b3):
    u_ids = U_ids.astype(jnp.int32)
    i_ids = I_ids.astype(jnp.int32)
    u_emb, i_emb = _sc_gather(u_ids, i_ids, user_table, item_table)
    return _mlp(u_emb, i_emb, W1[:D], W1[D:], b1.reshape(1, D),
                W2, b2.reshape(1, 8), W3.reshape(1, 8), b3.reshape(1, 1))


# speedup vs baseline: 1.4941x; 1.0012x over previous
import functools

import jax
import jax.numpy as jnp
from jax import lax
from jax.experimental import pallas as pl
from jax.experimental.pallas import tpu as pltpu
from jax.experimental.pallas import tpu_sc as plsc

BATCH = 16384
D = 32
NC = 2
NS = 16
NW = NC * NS
BPW = BATCH // NW
CH = 128
NCH = BPW // CH


def _gather_body(u_ids, i_ids, user_tab, item_tab, u_out, i_out,
                 idx_v, rows_a, rows_b, sem_a, sem_b):
    wid = lax.axis_index("s") * NC + lax.axis_index("c")
    base = wid * BPW
    pltpu.sync_copy(u_ids.at[pl.ds(base, BPW)], idx_v.at[0])
    pltpu.sync_copy(i_ids.at[pl.ds(base, BPW)], idx_v.at[1])

    bufs = (rows_a, rows_b)
    sems = (sem_a, sem_b)
    tabs = (user_tab, item_tab)
    outs = (u_out, i_out)

    def issue_chunk(step, slot):
        tab, ch = divmod(step, NCH)
        src = tabs[tab]
        buf = bufs[slot]
        sem = sems[slot]

        def body(g, _):
            vec = idx_v[tab, pl.ds(ch * CH + g * 16, 16)]
            off = g * 16
            for l in range(16):
                idx = vec[l]
                pltpu.make_async_copy(src.at[pl.ds(idx, 1)],
                                      buf.at[pl.ds(off + l, 1)], sem).start()
            return _

        lax.fori_loop(0, CH // 16, body, None)

    def drain_chunk(step, slot):
        tab, ch = divmod(step, NCH)
        pltpu.make_async_copy(tabs[tab].at[pl.ds(0, CH)], bufs[slot],
                              sems[slot]).wait()
        pltpu.sync_copy(bufs[slot],
                        outs[tab].at[pl.ds(base + ch * CH, CH)])

    for step in range(2 * NCH + 2):
        slot = step % 2
        if step >= 2:
            drain_chunk(step - 2, slot)
        if step < 2 * NCH:
            issue_chunk(step, slot)


_sc_gather = pl.kernel(
    _gather_body,
    out_type=(
        jax.ShapeDtypeStruct((BATCH, D), jnp.float32),
        jax.ShapeDtypeStruct((BATCH, D), jnp.float32),
    ),
    mesh=plsc.VectorSubcoreMesh(core_axis_name="c", subcore_axis_name="s"),
    scratch_types=[
        pltpu.VMEM((2, BPW), jnp.int32),
        pltpu.VMEM((CH, D), jnp.float32),
        pltpu.VMEM((CH, D), jnp.float32),
        pltpu.SemaphoreType.DMA,
        pltpu.SemaphoreType.DMA,
    ],
    compiler_params=pltpu.CompilerParams(use_tc_tiling_on_sc=True),
)


def _mlp_body(u_ref, i_ref, w1u_ref, w1i_ref, b1_ref, w2_ref, b2_ref,
              w3_ref, b3_ref, out_ref):
    u = u_ref[...]
    i = i_ref[...]
    h = jnp.dot(u, w1u_ref[...], preferred_element_type=jnp.float32)
    h = h + jnp.dot(i, w1i_ref[...], preferred_element_type=jnp.float32)
    h = jnp.maximum(h + b1_ref[...], 0.0)
    h2 = jnp.dot(h, w2_ref[...], preferred_element_type=jnp.float32)
    h2 = jnp.maximum(h2 + b2_ref[...], 0.0)
    out_ref[...] = jnp.sum(h2 * w3_ref[...], axis=1, keepdims=True) + b3_ref[...]


_mlp = pl.pallas_call(
    _mlp_body,
    out_shape=jax.ShapeDtypeStruct((BATCH, 1), jnp.float32),
)


def kernel(U_ids, I_ids, user_table, item_table, W1, b1, W2, b2, W3, b3):
    u_ids = U_ids.astype(jnp.int32)
    i_ids = I_ids.astype(jnp.int32)
    u_emb, i_emb = _sc_gather(u_ids, i_ids, user_table, item_table)
    return _mlp(u_emb, i_emb, W1[:D], W1[D:], b1.reshape(1, D),
                W2, b2.reshape(1, 8), W3.reshape(1, 8), b3.reshape(1, 1))
